# Initial kernel scaffold; baseline (speedup 1.0000x reference)
import jax
import jax.numpy as jnp
from jax.experimental import pallas as pl


def kernel(embeddings):
    raise NotImplementedError("write your pallas kernel here")


# speedup vs baseline: 12058.8735x; 12058.8735x over previous
import functools

import jax
import jax.numpy as jnp
from jax import lax
from jax.experimental import pallas as pl
from jax.experimental.pallas import tpu as pltpu
from jax.experimental.pallas import tpu_sc as plsc

N = 16777216
NBINS = 256
NC = 2
NS = 16
NW = NC * NS
L = 16
PER_W = N // NW
C = 16384
NCH = PER_W // C
SL = C // L
UNROLL = 4

_mesh = plsc.VectorSubcoreMesh(core_axis_name="c", subcore_axis_name="s")


@functools.partial(
    pl.kernel,
    mesh=_mesh,
    out_type=[
        jax.ShapeDtypeStruct((NW * L,), jnp.float32),
        jax.ShapeDtypeStruct((NW * L,), jnp.float32),
    ],
    scratch_types=[
        pltpu.VMEM((C,), jnp.float32),
        pltpu.VMEM((C,), jnp.float32),
        pltpu.VMEM((L,), jnp.float32),
        pltpu.VMEM((L,), jnp.float32),
        pltpu.SemaphoreType.DMA,
        pltpu.SemaphoreType.DMA,
    ],
)
def _minmax_k(x_hbm, mins_hbm, maxs_hbm, in0, in1, mnb, mxb, si0, si1):
    wid = lax.axis_index("s") * NC + lax.axis_index("c")
    base = wid * PER_W
    bufs = (in0, in1)
    sems = (si0, si1)
    pltpu.make_async_copy(x_hbm.at[pl.ds(base, C)], in0, si0).start()
    pltpu.make_async_copy(x_hbm.at[pl.ds(base + C, C)], in1, si1).start()

    def chunk_body(g2, carry):
        mn, mx = carry
        for b in range(2):
            g = g2 * 2 + b
            buf, sem = bufs[b], sems[b]
            pltpu.make_async_copy(x_hbm.at[pl.ds(base + g * C, C)], buf, sem).wait()

            def inner(i, c):
                imn, imx = c
                o = i * (L * UNROLL)
                for u in range(UNROLL):
                    v = buf[pl.ds(o + u * L, L)]
                    imn = jnp.minimum(imn, v)
                    imx = jnp.maximum(imx, v)
                return (imn, imx)

            mn, mx = lax.fori_loop(0, SL // UNROLL, inner, (mn, mx))
            nxt = g + 2

            @pl.when(nxt < NCH)
            def _():
                pltpu.make_async_copy(
                    x_hbm.at[pl.ds(base + nxt * C, C)], buf, sem
                ).start()
        return (mn, mx)

    init = (
        jnp.full((L,), jnp.inf, jnp.float32),
        jnp.full((L,), -jnp.inf, jnp.float32),
    )
    mn, mx = lax.fori_loop(0, NCH // 2, chunk_body, init)
    mnb[...] = mn
    mxb[...] = mx
    pltpu.sync_copy(mnb, mins_hbm.at[pl.ds(wid * L, L)])
    pltpu.sync_copy(mxb, maxs_hbm.at[pl.ds(wid * L, L)])


@functools.partial(
    pl.kernel,
    mesh=_mesh,
    out_type=jax.ShapeDtypeStruct((N,), jnp.int32),
    scratch_types=[
        pltpu.VMEM((NW * L,), jnp.float32),
        pltpu.VMEM((NW * L,), jnp.float32),
        pltpu.VMEM((2 * L,), jnp.float32),
        pltpu.VMEM((2 * L,), jnp.float32),
        pltpu.VMEM((C,), jnp.float32),
        pltpu.VMEM((C,), jnp.float32),
        pltpu.VMEM((C,), jnp.int32),
        pltpu.VMEM((C,), jnp.int32),
        pltpu.SemaphoreType.DMA,
        pltpu.SemaphoreType.DMA,
        pltpu.SemaphoreType.DMA,
        pltpu.SemaphoreType.DMA,
    ],
)
def _bucketize_k(
    x_hbm, mins_hbm, maxs_hbm, out_hbm,
    mins_v, maxs_v, dupm, dupx, in0, in1, o0, o1, si0, si1, so0, so1,
):
    wid = lax.axis_index("s") * NC + lax.axis_index("c")
    base = wid * PER_W
    ibufs, obufs = (in0, in1), (o0, o1)
    isems, osems = (si0, si1), (so0, so1)

    pltpu.make_async_copy(x_hbm.at[pl.ds(base, C)], in0, si0).start()
    pltpu.make_async_copy(x_hbm.at[pl.ds(base + C, C)], in1, si1).start()
    pltpu.sync_copy(mins_hbm, mins_v)
    pltpu.sync_copy(maxs_hbm, maxs_v)

    mn = mins_v[pl.ds(0, L)]
    mx = maxs_v[pl.ds(0, L)]
    for i in range(1, NW):
        mn = jnp.minimum(mn, mins_v[pl.ds(i * L, L)])
        mx = jnp.maximum(mx, maxs_v[pl.ds(i * L, L)])
    dupm[pl.ds(0, L)] = mn
    dupm[pl.ds(L, L)] = mn
    dupx[pl.ds(0, L)] = mx
    dupx[pl.ds(L, L)] = mx
    mnb, mxb = mn, mx
    for k in range(1, L):
        mnb = jnp.minimum(mnb, dupm[pl.ds(k, L)])
        mxb = jnp.maximum(mxb, dupx[pl.ds(k, L)])
    stepv = (mxb - mnb) * jnp.float32(1.0 / NBINS)
    invv = jnp.float32(1.0) / stepv
    cap = jnp.full((L,), NBINS - 1, jnp.int32)

    def chunk_body(g2, carry):
        for b in range(2):
            g = g2 * 2 + b
            ib, ob = ibufs[b], obufs[b]
            isem, osem = isems[b], osems[b]
            pltpu.make_async_copy(x_hbm.at[pl.ds(base + g * C, C)], ib, isem).wait()

            @pl.when(g >= 2)
            def _():
                pltpu.make_async_copy(
                    ob, out_hbm.at[pl.ds(base + (g - 2) * C, C)], osem
                ).wait()

            def inner(i, c):
                o = i * (L * UNROLL)
                for u in range(UNROLL):
                    v = ib[pl.ds(o + u * L, L)]
                    t = (v - mnb) * invv
                    k = jnp.minimum(t.astype(jnp.int32), cap)
                    ob[pl.ds(o + u * L, L)] = k
                return c

            lax.fori_loop(0, SL // UNROLL, inner, 0)
            pltpu.make_async_copy(
                ob, out_hbm.at[pl.ds(base + g * C, C)], osem
            ).start()
            nxt = g + 2

            @pl.when(nxt < NCH)
            def _():
                pltpu.make_async_copy(
                    x_hbm.at[pl.ds(base + nxt * C, C)], ib, isem
                ).start()
        return carry

    lax.fori_loop(0, NCH // 2, chunk_body, 0)
    pltpu.make_async_copy(
        o0, out_hbm.at[pl.ds(base + (NCH - 2) * C, C)], so0
    ).wait()
    pltpu.make_async_copy(
        o1, out_hbm.at[pl.ds(base + (NCH - 1) * C, C)], so1
    ).wait()


def kernel(embeddings):
    mins, maxs = _minmax_k(embeddings)
    return _bucketize_k(embeddings, mins, maxs)
